# initial kernel scaffold (unmeasured)
import jax
import jax.numpy as jnp
from jax import lax
from jax.experimental import pallas as pl
from jax.experimental.pallas import tpu as pltpu


def kernel(
    x,
):
    def body(*refs):
        pass

    out_shape = jax.ShapeDtypeStruct(..., jnp.float32)
    return pl.pallas_call(body, out_shape=out_shape)(...)


# baseline (device time: 23317 ns/iter reference)
import jax
import jax.numpy as jnp
from jax import lax
from jax.experimental import pallas as pl
from jax.experimental.pallas import tpu as pltpu

N_DEV = 16


def kernel(x):
    m, n = x.shape

    def body(x_ref, out_ref, totals_ref, send_sems, recv_sems):
        my = lax.axis_index("i")

        B = 128
        row = lax.broadcasted_iota(jnp.int32, (B, B), 0)
        col = lax.broadcasted_iota(jnp.int32, (B, B), 1)
        L = (row >= col).astype(jnp.float32)

        running = jnp.zeros((1, n), jnp.float32)
        for j in range(m // B):
            blk = x_ref[pl.ds(j * B, B), :]
            within = jnp.dot(L, blk, preferred_element_type=jnp.float32)
            out_ref[pl.ds(j * B, B), :] = within + running
            running = running + within[B - 1 : B, :]
        totals_ref[pl.ds(my, 1), :] = running

        barrier_sem = pltpu.get_barrier_semaphore()
        for d in range(N_DEV):
            pl.semaphore_signal(
                barrier_sem,
                inc=1,
                device_id=(d,),
                device_id_type=pl.DeviceIdType.MESH,
            )
        pl.semaphore_wait(barrier_sem, N_DEV)

        sends = []
        for k in range(1, N_DEV):
            target = lax.rem(my + k, N_DEV)
            rdma = pltpu.make_async_remote_copy(
                src_ref=totals_ref.at[pl.ds(my, 1)],
                dst_ref=totals_ref.at[pl.ds(my, 1)],
                send_sem=send_sems.at[k - 1],
                recv_sem=recv_sems.at[k - 1],
                device_id=(target,),
                device_id_type=pl.DeviceIdType.MESH,
            )
            rdma.start()
            sends.append(rdma)

        for k in range(1, N_DEV):
            src = lax.rem(my - k + N_DEV, N_DEV)
            recv = pltpu.make_async_remote_copy(
                src_ref=totals_ref.at[pl.ds(src, 1)],
                dst_ref=totals_ref.at[pl.ds(src, 1)],
                send_sem=send_sems.at[k - 1],
                recv_sem=recv_sems.at[k - 1],
                device_id=(src,),
                device_id_type=pl.DeviceIdType.MESH,
            )
            recv.wait_recv()

        for rdma in sends:
            rdma.wait_send()

        rows = lax.broadcasted_iota(jnp.int32, (N_DEV, n), 0)
        mask = rows < my
        offset = jnp.sum(
            jnp.where(mask, totals_ref[:, :], 0.0), axis=0, keepdims=True
        )
        out_ref[:, :] = out_ref[:, :] + offset

    return pl.pallas_call(
        body,
        out_shape=jax.ShapeDtypeStruct((m, n), jnp.float32),
        in_specs=[pl.BlockSpec(memory_space=pltpu.VMEM)],
        out_specs=pl.BlockSpec(memory_space=pltpu.VMEM),
        scratch_shapes=[
            pltpu.VMEM((N_DEV, n), jnp.float32),
            pltpu.SemaphoreType.DMA((N_DEV - 1,)),
            pltpu.SemaphoreType.DMA((N_DEV - 1,)),
        ],
        compiler_params=pltpu.CompilerParams(collective_id=0),
    )(x)


# device time: 22389 ns/iter; 1.0414x vs baseline; 1.0414x over previous
import jax
import jax.numpy as jnp
from jax import lax
from jax.experimental import pallas as pl
from jax.experimental.pallas import tpu as pltpu

N_DEV = 16


def kernel(x):
    m, n = x.shape

    def body(x_ref, out_ref, totals_ref, send_sems, recv_sems):
        my = lax.axis_index("i")

        totals_ref[pl.ds(my, 1), :] = jnp.sum(
            x_ref[:, :], axis=0, keepdims=True
        )

        barrier_sem = pltpu.get_barrier_semaphore()
        for d in range(N_DEV):
            pl.semaphore_signal(
                barrier_sem,
                inc=1,
                device_id=(d,),
                device_id_type=pl.DeviceIdType.MESH,
            )
        pl.semaphore_wait(barrier_sem, N_DEV)

        sends = []
        for k in range(1, N_DEV):
            target = lax.rem(my + k, N_DEV)
            rdma = pltpu.make_async_remote_copy(
                src_ref=totals_ref.at[pl.ds(my, 1)],
                dst_ref=totals_ref.at[pl.ds(my, 1)],
                send_sem=send_sems.at[k - 1],
                recv_sem=recv_sems.at[k - 1],
                device_id=(target,),
                device_id_type=pl.DeviceIdType.MESH,
            )
            rdma.start()
            sends.append(rdma)

        B = 128
        row = lax.broadcasted_iota(jnp.int32, (B, B), 0)
        col = lax.broadcasted_iota(jnp.int32, (B, B), 1)
        L = (row >= col).astype(jnp.float32)

        running = jnp.zeros((1, n), jnp.float32)
        for j in range(m // B):
            blk = x_ref[pl.ds(j * B, B), :]
            within = jnp.dot(L, blk, preferred_element_type=jnp.float32)
            out_ref[pl.ds(j * B, B), :] = within + running
            running = running + within[B - 1 : B, :]

        for k in range(1, N_DEV):
            src = lax.rem(my - k + N_DEV, N_DEV)
            recv = pltpu.make_async_remote_copy(
                src_ref=totals_ref.at[pl.ds(src, 1)],
                dst_ref=totals_ref.at[pl.ds(src, 1)],
                send_sem=send_sems.at[k - 1],
                recv_sem=recv_sems.at[k - 1],
                device_id=(src,),
                device_id_type=pl.DeviceIdType.MESH,
            )
            recv.wait_recv()

        for rdma in sends:
            rdma.wait_send()

        rows = lax.broadcasted_iota(jnp.int32, (N_DEV, n), 0)
        mask = rows < my
        offset = jnp.sum(
            jnp.where(mask, totals_ref[:, :], 0.0), axis=0, keepdims=True
        )
        out_ref[:, :] = out_ref[:, :] + offset

    return pl.pallas_call(
        body,
        out_shape=jax.ShapeDtypeStruct((m, n), jnp.float32),
        in_specs=[pl.BlockSpec(memory_space=pltpu.VMEM)],
        out_specs=pl.BlockSpec(memory_space=pltpu.VMEM),
        scratch_shapes=[
            pltpu.VMEM((N_DEV, n), jnp.float32),
            pltpu.SemaphoreType.DMA((N_DEV - 1,)),
            pltpu.SemaphoreType.DMA((N_DEV - 1,)),
        ],
        compiler_params=pltpu.CompilerParams(collective_id=0),
    )(x)


# device time: 22321 ns/iter; 1.0446x vs baseline; 1.0030x over previous
import jax
import jax.numpy as jnp
from jax import lax
from jax.experimental import pallas as pl
from jax.experimental.pallas import tpu as pltpu

N_DEV = 16


def kernel(x):
    m, n = x.shape

    def body(x_ref, out_ref, totals_ref, send_sems, recv_sems):
        my = lax.axis_index("i")

        totals_ref[pl.ds(my, 1), :] = jnp.sum(
            x_ref[:, :], axis=0, keepdims=True
        )

        barrier_sem = pltpu.get_barrier_semaphore()
        for d in range(N_DEV):
            pl.semaphore_signal(
                barrier_sem,
                inc=1,
                device_id=(d,),
                device_id_type=pl.DeviceIdType.MESH,
            )
        pl.semaphore_wait(barrier_sem, N_DEV)

        sends = []
        for k in range(1, N_DEV):
            target = lax.rem(my + k, N_DEV)
            rdma = pltpu.make_async_remote_copy(
                src_ref=totals_ref.at[pl.ds(my, 1)],
                dst_ref=totals_ref.at[pl.ds(my, 1)],
                send_sem=send_sems.at[k - 1],
                recv_sem=recv_sems.at[k - 1],
                device_id=(target,),
                device_id_type=pl.DeviceIdType.MESH,
            )
            rdma.start()
            sends.append(rdma)

        B = 128
        row = lax.broadcasted_iota(jnp.int32, (B, B), 0)
        col = lax.broadcasted_iota(jnp.int32, (B, B), 1)
        L = (row >= col).astype(jnp.bfloat16)

        running = jnp.zeros((1, n), jnp.float32)
        for j in range(m // B):
            blk = x_ref[pl.ds(j * B, B), :].astype(jnp.bfloat16)
            within = jnp.dot(L, blk, preferred_element_type=jnp.float32)
            out_ref[pl.ds(j * B, B), :] = within + running
            running = running + within[B - 1 : B, :]

        for k in range(1, N_DEV):
            src = lax.rem(my - k + N_DEV, N_DEV)
            recv = pltpu.make_async_remote_copy(
                src_ref=totals_ref.at[pl.ds(src, 1)],
                dst_ref=totals_ref.at[pl.ds(src, 1)],
                send_sem=send_sems.at[k - 1],
                recv_sem=recv_sems.at[k - 1],
                device_id=(src,),
                device_id_type=pl.DeviceIdType.MESH,
            )
            recv.wait_recv()

        for rdma in sends:
            rdma.wait_send()

        rows = lax.broadcasted_iota(jnp.int32, (N_DEV, n), 0)
        mask = rows < my
        offset = jnp.sum(
            jnp.where(mask, totals_ref[:, :], 0.0), axis=0, keepdims=True
        )
        out_ref[:, :] = out_ref[:, :] + offset

    return pl.pallas_call(
        body,
        out_shape=jax.ShapeDtypeStruct((m, n), jnp.float32),
        in_specs=[pl.BlockSpec(memory_space=pltpu.VMEM)],
        out_specs=pl.BlockSpec(memory_space=pltpu.VMEM),
        scratch_shapes=[
            pltpu.VMEM((N_DEV, n), jnp.float32),
            pltpu.SemaphoreType.DMA((N_DEV - 1,)),
            pltpu.SemaphoreType.DMA((N_DEV - 1,)),
        ],
        compiler_params=pltpu.CompilerParams(collective_id=0),
    )(x)
